# baseline (speedup 1.0000x reference)
import jax
import jax.numpy as jnp
import numpy as np
from jax import lax
from jax.experimental import pallas as pl
from jax.experimental.pallas import tpu as pltpu
from jax.experimental.pallas import tpu_sc as plsc

B, AT, NBR, NFEAT = 2, 512, 16, 128
NM = NBR - 1
NC, NS = 2, 16
NW = NC * NS
NATOMS = B * AT
APW = NATOMS // NW
WPB = AT // APW
RING = 3
IPA = NBR * NBR

_PAT = np.zeros((NBR, NBR), np.int32)
for _i in range(NBR):
    _PAT[_i, :NM] = np.delete(np.arange(NBR), _i)
_PAT_FLAT = np.ascontiguousarray(_PAT.reshape(-1))


def _sc_body(emb_hbm, nbr_hbm, pat_hbm, out_hbm, nbr_v, pat_v, idx_v, stg_v,
             gsem, wsem):
    wid = lax.axis_index("s") * NC + lax.axis_index("c")
    base = wid * APW
    bb = wid // WPB
    ab = (wid % WPB) * APW
    pltpu.sync_copy(pat_hbm, pat_v)
    pltpu.sync_copy(nbr_hbm.at[pl.ds(base * NBR, APW * NBR)], nbr_v)

    def build_idx(a, carry):
        off = a * NBR
        for i in range(NBR):
            vals = plsc.load_gather(nbr_v, [pat_v[pl.ds(i * NBR, NBR)] + off])
            idx_v[pl.ds(a * IPA + i * NBR, NBR)] = vals
        return carry

    lax.fori_loop(0, APW, build_idx, 0)

    def gathers(a, slot):
        for i in range(NBR):
            pltpu.async_copy(
                emb_hbm.at[idx_v.at[pl.ds(a * IPA + i * NBR, NM)]],
                stg_v.at[slot, i],
                gsem,
            )

    def wait_gathers(slot):
        for i in range(NBR):
            pltpu.make_async_copy(
                emb_hbm.at[idx_v.at[pl.ds(0, NM)]], stg_v.at[slot, i], gsem
            ).wait()

    def wait_write(slot):
        pltpu.make_async_copy(
            stg_v.at[slot], out_hbm.at[bb, ab], wsem
        ).wait()

    gathers(0, 0)
    gathers(1, 1)

    def atom_body(a, carry):
        slot = lax.rem(a, RING)
        wait_gathers(slot)
        pltpu.async_copy(stg_v.at[slot], out_hbm.at[bb, ab + a], wsem)

        @pl.when(a >= 1)
        def _():
            wait_write(lax.rem(a - 1, RING))

        @pl.when(a + 2 < APW)
        def _():
            gathers(a + 2, lax.rem(a + 2, RING))

        return carry

    lax.fori_loop(0, APW, atom_body, 0)
    wait_write(lax.rem(APW - 1, RING))


def kernel(node_embedding, nbr_idx):
    emb_flat = node_embedding.reshape(NATOMS, NFEAT)
    batch_off = (jnp.arange(B, dtype=jnp.int32) * AT)[:, None, None]
    nbr_glob = (nbr_idx.astype(jnp.int32) + batch_off).reshape(NATOMS * NBR)

    run = pl.kernel(
        _sc_body,
        out_type=jax.ShapeDtypeStruct((B, AT, NBR, NM, NFEAT), jnp.float32),
        mesh=plsc.VectorSubcoreMesh(core_axis_name="c", subcore_axis_name="s"),
        scratch_types=[
            pltpu.VMEM((APW * NBR,), jnp.int32),
            pltpu.VMEM((IPA,), jnp.int32),
            pltpu.VMEM((APW * IPA,), jnp.int32),
            pltpu.VMEM((RING, NBR, NM, NFEAT), jnp.float32),
            pltpu.SemaphoreType.DMA,
            pltpu.SemaphoreType.DMA,
        ],
        compiler_params=pltpu.CompilerParams(needs_layout_passes=False),
    )
    return run(emb_flat, nbr_glob, jnp.asarray(_PAT_FLAT))


# speedup vs baseline: 1.3039x; 1.3039x over previous
import jax
import jax.numpy as jnp
from jax import lax
from jax.experimental import pallas as pl
from jax.experimental.pallas import tpu as pltpu
from jax.experimental.pallas import tpu_sc as plsc

B, AT, NBR, NFEAT = 2, 512, 16, 128
NM = NBR - 1
NC, NS = 2, 16
NW = NC * NS
NATOMS = B * AT
APW = NATOMS // NW
CA = 8


def _sc_gather_body(emb_hbm, nbr_hbm, tbl_hbm, nbr_v, rows_v, gsem, wsem):
    wid = lax.axis_index("s") * NC + lax.axis_index("c")
    base = wid * APW
    pltpu.sync_copy(nbr_hbm.at[pl.ds(base, APW)], nbr_v)
    for a in range(APW):
        pltpu.async_copy(emb_hbm.at[nbr_v.at[a]], rows_v.at[a], gsem)
    for a in range(APW):
        pltpu.make_async_copy(emb_hbm.at[nbr_v.at[a]], rows_v.at[a], gsem).wait()
    pltpu.sync_copy(rows_v, tbl_hbm.at[pl.ds(base, APW)])


def _tc_expand_body(tbl_ref, out_ref):
    x = tbl_ref[...]
    for i in range(NBR):
        if i > 0:
            out_ref[:, i, : i, :] = x[:, :i, :]
        if i < NBR - 1:
            out_ref[:, i, i:, :] = x[:, i + 1 :, :]


def kernel(node_embedding, nbr_idx):
    emb_flat = node_embedding.reshape(NATOMS, NFEAT)
    batch_off = (jnp.arange(B, dtype=jnp.int32) * AT)[:, None, None]
    nbr_glob = (nbr_idx.astype(jnp.int32) + batch_off).reshape(NATOMS, NBR)

    gather = pl.kernel(
        _sc_gather_body,
        out_type=jax.ShapeDtypeStruct((NATOMS, NBR, NFEAT), jnp.float32),
        mesh=plsc.VectorSubcoreMesh(core_axis_name="c", subcore_axis_name="s"),
        scratch_types=[
            pltpu.VMEM((APW, NBR), jnp.int32),
            pltpu.VMEM((APW, NBR, NFEAT), jnp.float32),
            pltpu.SemaphoreType.DMA,
            pltpu.SemaphoreType.DMA,
        ],
        compiler_params=pltpu.CompilerParams(needs_layout_passes=False),
    )
    tbl = gather(emb_flat, nbr_glob)

    out = pl.pallas_call(
        _tc_expand_body,
        grid=(NATOMS // CA,),
        in_specs=[pl.BlockSpec((CA, NBR, NFEAT), lambda g: (g, 0, 0))],
        out_specs=pl.BlockSpec((CA, NBR, NM, NFEAT), lambda g: (g, 0, 0, 0)),
        out_shape=jax.ShapeDtypeStruct((NATOMS, NBR, NM, NFEAT), jnp.float32),
    )(tbl)
    return out.reshape(B, AT, NBR, NM, NFEAT)


# speedup vs baseline: 1.3563x; 1.0402x over previous
import jax
import jax.numpy as jnp
from jax import lax
from jax.experimental import pallas as pl
from jax.experimental.pallas import tpu as pltpu
from jax.experimental.pallas import tpu_sc as plsc

B, AT, NBR, NFEAT = 2, 512, 16, 128
NM = NBR - 1
NC, NS = 2, 16
NW = NC * NS
NATOMS = B * AT
APW = NATOMS // NW
CA = 8


def _sc_gather_body(emb_hbm, nbr_hbm, tbl_hbm, nbr_v, rows_v, gsem, wsem):
    wid = lax.axis_index("s") * NC + lax.axis_index("c")
    base = wid * APW
    pltpu.sync_copy(nbr_hbm.at[pl.ds(base, APW)], nbr_v)
    for a in range(APW):
        pltpu.async_copy(emb_hbm.at[nbr_v.at[a]], rows_v.at[a], gsem)
    for a in range(APW):
        pltpu.make_async_copy(emb_hbm.at[nbr_v.at[a]], rows_v.at[a], gsem).wait()
    pltpu.sync_copy(rows_v, tbl_hbm.at[pl.ds(base, APW)])


def _tc_expand_body(tbl_ref, out_ref):
    x = tbl_ref[...]
    for i in range(NBR):
        if i > 0:
            out_ref[0, :, i, : i, :] = x[0, :, :i, :]
        if i < NBR - 1:
            out_ref[0, :, i, i:, :] = x[0, :, i + 1 :, :]


def kernel(node_embedding, nbr_idx):
    emb_flat = node_embedding.reshape(NATOMS, NFEAT)
    batch_off = (jnp.arange(B, dtype=jnp.int32) * AT)[:, None, None]
    nbr_glob = (nbr_idx.astype(jnp.int32) + batch_off).reshape(NATOMS, NBR)

    gather = pl.kernel(
        _sc_gather_body,
        out_type=jax.ShapeDtypeStruct((NATOMS, NBR, NFEAT), jnp.float32),
        mesh=plsc.VectorSubcoreMesh(core_axis_name="c", subcore_axis_name="s"),
        scratch_types=[
            pltpu.VMEM((APW, NBR), jnp.int32),
            pltpu.VMEM((APW, NBR, NFEAT), jnp.float32),
            pltpu.SemaphoreType.DMA,
            pltpu.SemaphoreType.DMA,
        ],
        compiler_params=pltpu.CompilerParams(needs_layout_passes=False),
    )
    tbl = gather(emb_flat, nbr_glob)

    out = pl.pallas_call(
        _tc_expand_body,
        grid=(B, AT // CA),
        in_specs=[
            pl.BlockSpec((1, CA, NBR, NFEAT), lambda b, g: (b, g, 0, 0))
        ],
        out_specs=pl.BlockSpec(
            (1, CA, NBR, NM, NFEAT), lambda b, g: (b, g, 0, 0, 0)
        ),
        out_shape=jax.ShapeDtypeStruct((B, AT, NBR, NM, NFEAT), jnp.float32),
    )(tbl.reshape(B, AT, NBR, NFEAT))
    return out


# speedup vs baseline: 1.7216x; 1.2694x over previous
import jax
import jax.numpy as jnp
from jax import lax
from jax.experimental import pallas as pl
from jax.experimental.pallas import tpu as pltpu
from jax.experimental.pallas import tpu_sc as plsc

B, AT, NBR, NFEAT = 2, 512, 16, 128
NM = NBR - 1
RPA = NBR * NM
NC, NS = 2, 16
NW = NC * NS
NATOMS = B * AT
APW = NATOMS // NW
WPB = AT // APW


HALF = APW // 2


def _write_runs(out_hbm, rows_v, wsem, bb, ab, h, issue):
    copy = pltpu.async_copy if issue else (
        lambda s, d, m: pltpu.make_async_copy(s, d, m).wait()
    )
    a0 = h * HALF
    for i in range(NBR):
        if i > 0:
            copy(
                rows_v.at[pl.ds(a0, HALF), pl.ds(0, i)],
                out_hbm.at[bb, pl.ds(ab + a0, HALF), i, pl.ds(0, i)],
                wsem,
            )
        if i < NBR - 1:
            copy(
                rows_v.at[pl.ds(a0, HALF), pl.ds(i + 1, NM - i)],
                out_hbm.at[bb, pl.ds(ab + a0, HALF), i, pl.ds(i, NM - i)],
                wsem,
            )


def _sc_body(emb_hbm, nbr_hbm, out_hbm, nbr_v, rows_v, gsem, wsem):
    wid = lax.axis_index("s") * NC + lax.axis_index("c")
    base = wid * APW
    bb = wid // WPB
    ab = (wid % WPB) * APW
    pltpu.sync_copy(nbr_hbm.at[pl.ds(base, APW)], nbr_v)
    for a in range(APW):
        pltpu.async_copy(emb_hbm.at[nbr_v.at[a]], rows_v.at[a], gsem)
    for h in range(2):
        for a in range(h * HALF, (h + 1) * HALF):
            pltpu.make_async_copy(
                emb_hbm.at[nbr_v.at[a]], rows_v.at[a], gsem
            ).wait()
        _write_runs(out_hbm, rows_v, wsem, bb, ab, h, True)
    for h in range(2):
        _write_runs(out_hbm, rows_v, wsem, bb, ab, h, False)


def kernel(node_embedding, nbr_idx):
    emb_flat = node_embedding.reshape(NATOMS, NFEAT)
    batch_off = (jnp.arange(B, dtype=jnp.int32) * AT)[:, None, None]
    nbr_glob = (nbr_idx.astype(jnp.int32) + batch_off).reshape(NATOMS, NBR)

    run = pl.kernel(
        _sc_body,
        out_type=jax.ShapeDtypeStruct((B, AT, NBR, NBR, NFEAT), jnp.float32),
        mesh=plsc.VectorSubcoreMesh(core_axis_name="c", subcore_axis_name="s"),
        scratch_types=[
            pltpu.VMEM((APW, NBR), jnp.int32),
            pltpu.VMEM((APW, NBR, NFEAT), jnp.float32),
            pltpu.SemaphoreType.DMA,
            pltpu.SemaphoreType.DMA,
        ],
        compiler_params=pltpu.CompilerParams(
            needs_layout_passes=False, use_tc_tiling_on_sc=False
        ),
    )
    return run(emb_flat, nbr_glob)[:, :, :, :NM, :]


# speedup vs baseline: 1.7865x; 1.0377x over previous
import jax
import jax.numpy as jnp
from jax import lax
from jax.experimental import pallas as pl
from jax.experimental.pallas import tpu as pltpu
from jax.experimental.pallas import tpu_sc as plsc

B, AT, NBR, NFEAT = 2, 512, 16, 128
NM = NBR - 1
RPA = NBR * NM
NC, NS = 2, 16
NW = NC * NS
NATOMS = B * AT
APW = NATOMS // NW
WPB = AT // APW


def _write_runs(out_hbm, rows_v, wsem, bb, ab, issue):
    copy = pltpu.async_copy if issue else (
        lambda s, d, m: pltpu.make_async_copy(s, d, m).wait()
    )
    for i in range(NBR):
        if i > 0:
            copy(
                rows_v.at[:, pl.ds(0, i)],
                out_hbm.at[bb, pl.ds(ab, APW), i, pl.ds(0, i)],
                wsem,
            )
        if i < NBR - 1:
            copy(
                rows_v.at[:, pl.ds(i + 1, NM - i)],
                out_hbm.at[bb, pl.ds(ab, APW), i, pl.ds(i, NM - i)],
                wsem,
            )


def _sc_body(emb_hbm, nbr_hbm, out_hbm, nbr_v, rows_v, gsem, wsem):
    wid = lax.axis_index("s") * NC + lax.axis_index("c")
    base = wid * APW
    bb = wid // WPB
    ab = (wid % WPB) * APW
    pltpu.sync_copy(nbr_hbm.at[pl.ds(base, APW)], nbr_v)
    for a in range(APW):
        pltpu.async_copy(emb_hbm.at[nbr_v.at[a]], rows_v.at[a], gsem)
    for a in range(APW):
        pltpu.make_async_copy(emb_hbm.at[nbr_v.at[a]], rows_v.at[a], gsem).wait()
    _write_runs(out_hbm, rows_v, wsem, bb, ab, True)
    _write_runs(out_hbm, rows_v, wsem, bb, ab, False)


def kernel(node_embedding, nbr_idx):
    emb_flat = node_embedding.reshape(NATOMS, NFEAT)
    batch_off = (jnp.arange(B, dtype=jnp.int32) * AT)[:, None, None]
    nbr_glob = (nbr_idx.astype(jnp.int32) + batch_off).reshape(NATOMS, NBR)

    run = pl.kernel(
        _sc_body,
        out_type=jax.ShapeDtypeStruct((B, AT, NBR, NBR, NFEAT), jnp.float32),
        mesh=plsc.VectorSubcoreMesh(core_axis_name="c", subcore_axis_name="s"),
        scratch_types=[
            pltpu.VMEM((APW, NBR), jnp.int32),
            pltpu.VMEM((APW, NBR, NFEAT), jnp.float32),
            pltpu.SemaphoreType.DMA,
            pltpu.SemaphoreType.DMA,
        ],
        compiler_params=pltpu.CompilerParams(
            needs_layout_passes=False, use_tc_tiling_on_sc=False
        ),
    )
    return run(emb_flat, nbr_glob)[:, :, :, :NM, :]
